# baseline (speedup 1.0000x reference)
import functools

import jax
import jax.numpy as jnp
from jax import lax
from jax.experimental import pallas as pl
from jax.experimental.pallas import tpu as pltpu
from jax.experimental.pallas import tpu_sc as plsc


@functools.lru_cache(maxsize=None)
def _make_gather(B, S, V2, D):
    info = plsc.get_sparse_core_info()
    NC, NS = info.num_cores, info.num_subcores
    NW = NC * NS
    assert B % NW == 0
    BW = B // NW
    mesh = plsc.VectorSubcoreMesh(core_axis_name="c", subcore_axis_name="s")

    @functools.partial(
        pl.kernel,
        mesh=mesh,
        compiler_params=pltpu.CompilerParams(use_tc_tiling_on_sc=False),
        out_type=jax.ShapeDtypeStruct((B, S, D), jnp.float32),
        scratch_types=[
            pltpu.VMEM((8, S), jnp.int32),
            pltpu.VMEM((8, S, D), jnp.float32),
            pltpu.SemaphoreType.DMA((8,)),
            pltpu.SemaphoreType.DMA((8,)),
            pltpu.SemaphoreType.DMA((8,)),
        ],
    )
    def gather_kernel(tok_hbm, table_hbm, out_hbm, idx_v, rows_v,
                      sem_i, sem_g, sem_o):
        K = 8
        L = 6
        wid = lax.axis_index("s") * NC + lax.axis_index("c")
        b0 = wid * BW

        def idx_load(r, slot):
            return pltpu.make_async_copy(
                tok_hbm.at[b0 + r], idx_v.at[slot], sem_i.at[slot])

        def row_gather(slot):
            return pltpu.make_async_copy(
                table_hbm.at[idx_v.at[slot]], rows_v.at[slot], sem_g.at[slot])

        def writeback(r, slot):
            return pltpu.make_async_copy(
                rows_v.at[slot], out_hbm.at[b0 + r], sem_o.at[slot])

        for k in range(K):
            idx_load(k, k).start()

        def body(r, carry):
            @pl.when(r < BW)
            def _():
                slot = r % K

                @pl.when(r >= K)
                def _():
                    writeback(r - K, slot).wait()
                idx_load(r, slot).wait()
                row_gather(slot).start()

            @pl.when(r >= L)
            def _():
                g = r - L
                gs = g % K
                row_gather(gs).wait()
                writeback(g, gs).start()

                @pl.when(g + K < BW)
                def _():
                    idx_load(g + K, gs).start()
            return carry

        lax.fori_loop(0, BW + L, body, 0, unroll=False)

        for j in range(BW - K, BW):
            writeback(j, j % K).wait()

    return gather_kernel


def kernel(token_ids, model_idx, table_0):
    B, S = token_ids.shape
    V, D = table_0.shape
    table_p = jnp.pad(table_0, ((0, 0), (0, 128 - D))).reshape(2 * V, D)
    tok2 = token_ids * 2
    o = _make_gather(B, S, 2 * V, D)(tok2, table_p)
    return o


# speedup vs baseline: 1.0014x; 1.0014x over previous
import functools

import jax
import jax.numpy as jnp
from jax import lax
from jax.experimental import pallas as pl
from jax.experimental.pallas import tpu as pltpu
from jax.experimental.pallas import tpu_sc as plsc


@functools.lru_cache(maxsize=None)
def _make_gather(B, S, V2, D):
    info = plsc.get_sparse_core_info()
    NC, NS = info.num_cores, info.num_subcores
    NW = NC * NS
    assert B % NW == 0
    BW = B // NW
    mesh = plsc.VectorSubcoreMesh(core_axis_name="c", subcore_axis_name="s")

    @functools.partial(
        pl.kernel,
        mesh=mesh,
        compiler_params=pltpu.CompilerParams(use_tc_tiling_on_sc=False),
        out_type=jax.ShapeDtypeStruct((B, S, D), jnp.float32),
        scratch_types=[
            pltpu.VMEM((8, S), jnp.int32),
            pltpu.VMEM((8, S, D), jnp.float32),
            pltpu.SemaphoreType.DMA((8,)),
            pltpu.SemaphoreType.DMA((8,)),
            pltpu.SemaphoreType.DMA((8,)),
        ],
    )
    def gather_kernel(tok_hbm, table_hbm, out_hbm, idx_v, rows_v,
                      sem_i, sem_g, sem_o):
        K = 8
        L = 4
        wid = lax.axis_index("s") * NC + lax.axis_index("c")
        b0 = wid * BW

        def idx_load(r, slot):
            return pltpu.make_async_copy(
                tok_hbm.at[b0 + r], idx_v.at[slot], sem_i.at[slot])

        def row_gather(slot):
            return pltpu.make_async_copy(
                table_hbm.at[idx_v.at[slot]], rows_v.at[slot], sem_g.at[slot])

        def writeback(r, slot):
            return pltpu.make_async_copy(
                rows_v.at[slot], out_hbm.at[b0 + r], sem_o.at[slot])

        for k in range(K):
            idx_load(k, k).start()

        def body(r, carry):
            @pl.when(r < BW)
            def _():
                slot = r % K

                @pl.when(r >= K)
                def _():
                    writeback(r - K, slot).wait()
                idx_load(r, slot).wait()
                row_gather(slot).start()

            @pl.when(r >= L)
            def _():
                g = r - L
                gs = g % K
                row_gather(gs).wait()
                writeback(g, gs).start()

                @pl.when(g + K < BW)
                def _():
                    idx_load(g + K, gs).start()
            return carry

        lax.fori_loop(0, BW + L, body, 0, unroll=False)

        for j in range(BW - K, BW):
            writeback(j, j % K).wait()

    return gather_kernel


def kernel(token_ids, model_idx, table_0):
    B, S = token_ids.shape
    V, D = table_0.shape
    table_p = jnp.pad(table_0, ((0, 0), (0, 128 - D))).reshape(2 * V, D)
    tok2 = token_ids * 2
    o = _make_gather(B, S, 2 * V, D)(tok2, table_p)
    return o
